# baseline (speedup 1.0000x reference)
import jax
import jax.numpy as jnp
from jax.experimental import pallas as pl
from jax.experimental.pallas import tpu as pltpu

N_CLASSES = 1000
N_CTX = 8
NAME_LEN = 77
OUT_LEN = N_CTX + NAME_LEN
CTX_DIM = 512

CHUNK = 50
NSTEPS = N_CLASSES // CHUNK
NBUF = 4


def _ring_body(ctx_hbm, name_hbm, out_hbm, obuf, isems, osems):
    def in_copies(step):
        slot = step % NBUF
        lo = step * CHUNK
        c1 = pltpu.make_async_copy(
            ctx_hbm.at[pl.ds(lo, CHUNK)],
            obuf.at[slot, :, pl.ds(0, N_CTX), :],
            isems.at[slot, 0],
        )
        c2 = pltpu.make_async_copy(
            name_hbm.at[pl.ds(lo, CHUNK)],
            obuf.at[slot, :, pl.ds(N_CTX, NAME_LEN), :],
            isems.at[slot, 1],
        )
        return c1, c2

    def out_copy(step):
        slot = step % NBUF
        return pltpu.make_async_copy(
            obuf.at[slot],
            out_hbm.at[pl.ds(step * CHUNK, CHUNK)],
            osems.at[slot],
        )

    ins = [None] * NSTEPS
    outs = [None] * NSTEPS

    def start_in(step):
        ins[step] = in_copies(step)
        ins[step][0].start()
        ins[step][1].start()

    waited_out = [False] * NSTEPS
    for step in range(min(NBUF - 1, NSTEPS)):
        start_in(step)
    for step in range(NSTEPS):
        ins[step][0].wait()
        ins[step][1].wait()
        outs[step] = out_copy(step)
        outs[step].start()
        nxt = step + NBUF - 1
        if nxt < NSTEPS:
            if step >= 1:
                outs[step - 1].wait()
                waited_out[step - 1] = True
            start_in(nxt)
    for step in range(NSTEPS):
        if outs[step] is not None and not waited_out[step]:
            outs[step].wait()


def kernel(ctx, name_embeds):
    return pl.pallas_call(
        _ring_body,
        in_specs=[
            pl.BlockSpec(memory_space=pl.ANY),
            pl.BlockSpec(memory_space=pl.ANY),
        ],
        out_specs=pl.BlockSpec(memory_space=pl.ANY),
        out_shape=jax.ShapeDtypeStruct((N_CLASSES, OUT_LEN, CTX_DIM), jnp.float32),
        scratch_shapes=[
            pltpu.VMEM((NBUF, CHUNK, OUT_LEN, CTX_DIM), jnp.float32),
            pltpu.SemaphoreType.DMA((NBUF, 2)),
            pltpu.SemaphoreType.DMA((NBUF,)),
        ],
    )(ctx, name_embeds)
